# Initial kernel scaffold; baseline (speedup 1.0000x reference)
import jax
import jax.numpy as jnp
from jax import lax
from jax.experimental import pallas as pl
from jax.experimental.pallas import tpu as pltpu
from jax.experimental.pallas import tpu_sc as plsc

B, C, H, W = 8, 16, 512, 512
P = H * W
NC, NS = 2, 16
NPT = P // NS
CP = 2048
NCH = NPT // CP
NV = CP // 16

PROWS = 516
PCOLS = 528
PLANE = PROWS * PCOLS
ZCHUNK = 17032
ZLAST = PLANE - (NS - 1) * ZCHUNK

F32 = jnp.float32
I32 = jnp.int32


def _splat_body(frame, flow, out, pbase, pw0, pw1, pw2, pw3, plane,
                fxb, fyb, valb, baseb, wb0, wb1, wb2, wb3,
                idxb, updb, zbuf):
    cid = lax.axis_index("c")
    sid = lax.axis_index("s")
    tile_px = sid * NPT
    lane_f = lax.iota(I32, 16).astype(F32)

    zbuf[...] = jnp.zeros_like(zbuf)

    def zero_plane():
        @pl.when(sid < NS - 1)
        def _():
            pltpu.sync_copy(zbuf.at[pl.ds(0, ZCHUNK)],
                            plane.at[pl.ds(sid * ZCHUNK, ZCHUNK)])

        @pl.when(sid == NS - 1)
        def _():
            pltpu.sync_copy(zbuf.at[pl.ds(0, ZLAST)],
                            plane.at[pl.ds((NS - 1) * ZCHUNK, ZLAST)])

    zero_plane()
    plsc.subcore_barrier()

    def one_image(ib, carry):
        b = cid * (B // NC) + ib

        def prep_chunk(ch, carry):
            pstart = tile_px + ch * CP
            pltpu.sync_copy(flow.at[b, 0, pl.ds(pstart, CP)], fxb)
            pltpu.sync_copy(flow.at[b, 1, pl.ds(pstart, CP)], fyb)

            def vreg(j, carry):
                p0 = pstart + j * 16
                xs = (p0 % W).astype(F32) + lane_f
                ys = jnp.full((16,), 1.0, F32) * (p0 // W).astype(F32)
                tx = jnp.clip(xs + fxb[pl.ds(j * 16, 16)], -3.0, 513.0)
                ty = jnp.clip(ys + fyb[pl.ds(j * 16, 16)], -3.0, 513.0)
                x0 = tx.astype(I32)
                y0 = ty.astype(I32)
                x0 = jnp.where(x0.astype(F32) > tx, x0 - 1, x0)
                y0 = jnp.where(y0.astype(F32) > ty, y0 - 1, y0)
                fx = tx - x0.astype(F32)
                fy = ty - y0.astype(F32)
                gx = 1.0 - fx
                gy = 1.0 - fy
                x0c = jnp.clip(x0, -2, 512)
                y0c = jnp.clip(y0, -2, 512)
                baseb[pl.ds(j * 16, 16)] = (y0c + 2) * PCOLS + (x0c + 8)
                wb0[pl.ds(j * 16, 16)] = gx * gy
                wb1[pl.ds(j * 16, 16)] = fx * gy
                wb2[pl.ds(j * 16, 16)] = gx * fy
                wb3[pl.ds(j * 16, 16)] = fx * fy
                return carry

            lax.fori_loop(0, NV, vreg, None, unroll=False)
            pltpu.sync_copy(baseb, pbase.at[pl.ds(pstart, CP)])
            pltpu.sync_copy(wb0, pw0.at[pl.ds(pstart, CP)])
            pltpu.sync_copy(wb1, pw1.at[pl.ds(pstart, CP)])
            pltpu.sync_copy(wb2, pw2.at[pl.ds(pstart, CP)])
            pltpu.sync_copy(wb3, pw3.at[pl.ds(pstart, CP)])
            return carry

        lax.fori_loop(0, NCH, prep_chunk, None, unroll=False)

        def one_channel(c, carry):
            def sc_chunk(ch, carry):
                pstart = tile_px + ch * CP
                pltpu.sync_copy(frame.at[b, c, pl.ds(pstart, CP)], valb)
                pltpu.sync_copy(pbase.at[pl.ds(pstart, CP)], baseb)
                pltpu.sync_copy(pw0.at[pl.ds(pstart, CP)], wb0)
                pltpu.sync_copy(pw1.at[pl.ds(pstart, CP)], wb1)
                pltpu.sync_copy(pw2.at[pl.ds(pstart, CP)], wb2)
                pltpu.sync_copy(pw3.at[pl.ds(pstart, CP)], wb3)

                def vreg(j, carry):
                    r = j >> 3
                    cc = (j & 7) * 16
                    v = valb[pl.ds(j * 16, 16)]
                    bs = baseb[pl.ds(j * 16, 16)]
                    idxb[0, r, pl.ds(cc, 16)] = bs
                    idxb[1, r, pl.ds(cc, 16)] = bs + 1
                    idxb[2, r, pl.ds(cc, 16)] = bs + PCOLS
                    idxb[3, r, pl.ds(cc, 16)] = bs + (PCOLS + 1)
                    updb[0, r, pl.ds(cc, 16)] = v * wb0[pl.ds(j * 16, 16)]
                    updb[1, r, pl.ds(cc, 16)] = v * wb1[pl.ds(j * 16, 16)]
                    updb[2, r, pl.ds(cc, 16)] = v * wb2[pl.ds(j * 16, 16)]
                    updb[3, r, pl.ds(cc, 16)] = v * wb3[pl.ds(j * 16, 16)]
                    return carry

                lax.fori_loop(0, NV, vreg, None, unroll=False)
                for k in range(4):
                    pltpu.sync_copy(updb.at[k], plane.at[idxb.at[k]], add=True)
                return carry

            lax.fori_loop(0, NCH, sc_chunk, None, unroll=False)
            plsc.subcore_barrier()

            def drain_row(r, carry):
                row = sid * (H // NS) + r
                pltpu.sync_copy(
                    plane.at[pl.ds((row + 2) * PCOLS + 8, W)],
                    out.at[b, c, pl.ds(row * W, W)])
                return carry

            lax.fori_loop(0, H // NS, drain_row, None, unroll=False)
            plsc.subcore_barrier()

            zero_plane()
            plsc.subcore_barrier()
            return carry

        lax.fori_loop(0, C, one_channel, None, unroll=False)
        return carry

    lax.fori_loop(0, B // NC, one_image, None, unroll=False)


def kernel(frame, flow):
    frame2 = frame.reshape(B, C, P)
    flow2 = flow.reshape(B, 2, P)
    mesh = plsc.VectorSubcoreMesh(core_axis_name="c", subcore_axis_name="s")
    fn = pl.kernel(
        _splat_body,
        out_type=jax.ShapeDtypeStruct((B, C, P), F32),
        mesh=mesh,
        scratch_types=[
            pltpu.VMEM_SHARED((P,), I32),
            pltpu.VMEM_SHARED((P,), F32),
            pltpu.VMEM_SHARED((P,), F32),
            pltpu.VMEM_SHARED((P,), F32),
            pltpu.VMEM_SHARED((P,), F32),
            pltpu.VMEM_SHARED((PLANE,), F32),
            pltpu.VMEM((CP,), F32),
            pltpu.VMEM((CP,), F32),
            pltpu.VMEM((CP,), F32),
            pltpu.VMEM((CP,), I32),
            pltpu.VMEM((CP,), F32),
            pltpu.VMEM((CP,), F32),
            pltpu.VMEM((CP,), F32),
            pltpu.VMEM((CP,), F32),
            pltpu.VMEM((4, NV // 8, 128), I32),
            pltpu.VMEM((4, NV // 8, 128), F32),
            pltpu.VMEM((ZCHUNK,), F32),
        ],
    )
    out = fn(frame2, flow2)
    return out.reshape(B, C, H, W)


# speedup vs baseline: 4.8465x; 4.8465x over previous
import jax
import jax.numpy as jnp
from jax import lax
from jax.experimental import pallas as pl
from jax.experimental.pallas import tpu as pltpu
from jax.experimental.pallas import tpu_sc as plsc

B, C, H, W = 8, 16, 512, 512
P = H * W
NC, NS = 2, 16
NPT = P // NS
CP = 2048
NCH = NPT // CP
NV = CP // 16

PROWS = 516
PCOLS = 768
XOFF = 128
PLANE = PROWS * PCOLS
ZCHUNK = 24832
ZLAST = PLANE - (NS - 1) * ZCHUNK

F32 = jnp.float32
I32 = jnp.int32


def _splat_body(frame, flow, out, pbase, pfx, pfy, plane,
                fxb, fyb, valb, baseb, wb0, wb1,
                idx0, idx1, idx2, idx3, upd0, upd1, upd2, upd3, zbuf):
    cid = lax.axis_index("c")
    sid = lax.axis_index("s")
    tile_px = sid * NPT
    lane_f = lax.iota(I32, 16).astype(F32)

    zbuf[...] = jnp.zeros_like(zbuf)

    def zero_plane():
        @pl.when(sid < NS - 1)
        def _():
            pltpu.sync_copy(zbuf.at[pl.ds(0, ZCHUNK)],
                            plane.at[pl.ds(sid * ZCHUNK, ZCHUNK)])

        @pl.when(sid == NS - 1)
        def _():
            pltpu.sync_copy(zbuf.at[pl.ds(0, ZLAST)],
                            plane.at[pl.ds((NS - 1) * ZCHUNK, ZLAST)])

    zero_plane()
    plsc.subcore_barrier()

    def one_image(ib, carry):
        b = cid * (B // NC) + ib

        def prep_chunk(ch, carry):
            pstart = tile_px + ch * CP
            pltpu.sync_copy(flow.at[b, 0, pl.ds(pstart, CP)], fxb)
            pltpu.sync_copy(flow.at[b, 1, pl.ds(pstart, CP)], fyb)

            def vreg(j, carry):
                p0 = pstart + j * 16
                xs = (p0 % W).astype(F32) + lane_f
                ys = jnp.full((16,), 1.0, F32) * (p0 // W).astype(F32)
                tx = jnp.clip(xs + fxb[pl.ds(j * 16, 16)], -3.0, 513.0)
                ty = jnp.clip(ys + fyb[pl.ds(j * 16, 16)], -3.0, 513.0)
                x0 = tx.astype(I32)
                y0 = ty.astype(I32)
                x0 = jnp.where(x0.astype(F32) > tx, x0 - 1, x0)
                y0 = jnp.where(y0.astype(F32) > ty, y0 - 1, y0)
                fx = tx - x0.astype(F32)
                fy = ty - y0.astype(F32)
                x0c = jnp.clip(x0, -2, 512)
                y0c = jnp.clip(y0, -2, 512)
                baseb[pl.ds(j * 16, 16)] = (y0c + 2) * PCOLS + (x0c + XOFF)
                wb0[pl.ds(j * 16, 16)] = fx
                wb1[pl.ds(j * 16, 16)] = fy
                return carry

            lax.fori_loop(0, NV, vreg, None, unroll=False)
            pltpu.sync_copy(baseb, pbase.at[pl.ds(pstart, CP)])
            pltpu.sync_copy(wb0, pfx.at[pl.ds(pstart, CP)])
            pltpu.sync_copy(wb1, pfy.at[pl.ds(pstart, CP)])
            return carry

        lax.fori_loop(0, NCH, prep_chunk, None, unroll=False)

        def one_channel(c, carry):
            def sc_chunk(ch, carry):
                pstart = tile_px + ch * CP
                pltpu.sync_copy(frame.at[b, c, pl.ds(pstart, CP)], valb)
                pltpu.sync_copy(pbase.at[pl.ds(pstart, CP)], baseb)
                pltpu.sync_copy(pfx.at[pl.ds(pstart, CP)], wb0)
                pltpu.sync_copy(pfy.at[pl.ds(pstart, CP)], wb1)

                def vreg(j, carry):
                    cc = j * 16
                    v = valb[pl.ds(cc, 16)]
                    bs = baseb[pl.ds(cc, 16)]
                    fx = wb0[pl.ds(cc, 16)]
                    fy = wb1[pl.ds(cc, 16)]
                    vgy = v - v * fy
                    vfy = v * fy
                    idx0[pl.ds(cc, 16)] = bs
                    idx1[pl.ds(cc, 16)] = bs + 1
                    idx2[pl.ds(cc, 16)] = bs + PCOLS
                    idx3[pl.ds(cc, 16)] = bs + (PCOLS + 1)
                    upd0[pl.ds(cc, 16)] = vgy - vgy * fx
                    upd1[pl.ds(cc, 16)] = vgy * fx
                    upd2[pl.ds(cc, 16)] = vfy - vfy * fx
                    upd3[pl.ds(cc, 16)] = vfy * fx
                    return carry

                lax.fori_loop(0, NV, vreg, None, unroll=False)
                pltpu.sync_copy(upd0, plane.at[idx0], add=True)
                pltpu.sync_copy(upd1, plane.at[idx1], add=True)
                pltpu.sync_copy(upd2, plane.at[idx2], add=True)
                pltpu.sync_copy(upd3, plane.at[idx3], add=True)
                return carry

            lax.fori_loop(0, NCH, sc_chunk, None, unroll=False)
            plsc.subcore_barrier()

            def drain_row(r, carry):
                row = sid * (H // NS) + r
                src_off = pl.multiple_of((row + 2) * PCOLS + XOFF, 128)
                dst_off = pl.multiple_of(row * W, W)
                pltpu.sync_copy(
                    plane.at[pl.ds(src_off, W)],
                    out.at[b, c, pl.ds(dst_off, W)])
                return carry

            lax.fori_loop(0, H // NS, drain_row, None, unroll=False)
            plsc.subcore_barrier()

            zero_plane()
            plsc.subcore_barrier()
            return carry

        lax.fori_loop(0, C, one_channel, None, unroll=False)
        return carry

    lax.fori_loop(0, B // NC, one_image, None, unroll=False)


def kernel(frame, flow):
    frame2 = frame.reshape(B, C, P)
    flow2 = flow.reshape(B, 2, P)
    mesh = plsc.VectorSubcoreMesh(core_axis_name="c", subcore_axis_name="s")
    fn = pl.kernel(
        _splat_body,
        out_type=jax.ShapeDtypeStruct((B, C, P), F32),
        mesh=mesh,
        scratch_types=[
            pltpu.VMEM_SHARED((P,), I32),
            pltpu.VMEM_SHARED((P,), F32),
            pltpu.VMEM_SHARED((P,), F32),
            pltpu.VMEM_SHARED((PLANE,), F32),
            pltpu.VMEM((CP,), F32),
            pltpu.VMEM((CP,), F32),
            pltpu.VMEM((CP,), F32),
            pltpu.VMEM((CP,), I32),
            pltpu.VMEM((CP,), F32),
            pltpu.VMEM((CP,), F32),
            pltpu.VMEM((CP,), I32),
            pltpu.VMEM((CP,), I32),
            pltpu.VMEM((CP,), I32),
            pltpu.VMEM((CP,), I32),
            pltpu.VMEM((CP,), F32),
            pltpu.VMEM((CP,), F32),
            pltpu.VMEM((CP,), F32),
            pltpu.VMEM((CP,), F32),
            pltpu.VMEM((ZCHUNK,), F32),
        ],
    )
    out = fn(frame2, flow2)
    return out.reshape(B, C, H, W)


# speedup vs baseline: 9.7987x; 2.0218x over previous
import jax
import jax.numpy as jnp
from jax import lax
from jax.experimental import pallas as pl
from jax.experimental.pallas import tpu as pltpu
from jax.experimental.pallas import tpu_sc as plsc

B, C, H, W = 8, 16, 512, 512
P = H * W
NC, NS = 2, 16
NPT = P // NS
CP = 2048
NCH = NPT // CP
NV = CP // 16

PROWS = 516
PCOLS = 768
XOFF = 128
PLANE = PROWS * PCOLS
ZCHUNK = 24832
ZLAST = PLANE - (NS - 1) * ZCHUNK
ROWS_PT = H // NS

F32 = jnp.float32
I32 = jnp.int32


def _splat_body(frame, flow, out, plane,
                fxr, fyr, ir0,
                val0, val1,
                ua0, ua1, ua2, ua3, ub0, ub1, ub2, ub3,
                ja1, ja2, ja3, jb1, jb2, jb3,
                sv0, sv1, ssc0, ssc1, smisc):
    cid = lax.axis_index("c")
    sid = lax.axis_index("s")
    tile_px = sid * NPT
    lane_f = lax.iota(I32, 16).astype(F32)

    vals = (val0, val1)
    svs = (sv0, sv1)
    upds = ((ua0, ua1, ua2, ua3), (ub0, ub1, ub2, ub3))
    jdx = ((ja1, ja2, ja3), (jb1, jb2, jb3))
    sscs = (ssc0, ssc1)

    ZB = 2048
    zbuf = ua0

    def zero_plane():
        def zb_init(j, carry):
            zbuf[pl.ds(j * 16, 16)] = jnp.zeros((16,), F32)
            return carry

        lax.fori_loop(0, ZB // 16, zb_init, None, unroll=False)

        @pl.when(sid < NS - 1)
        def _():
            ds = [pltpu.async_copy(
                zbuf, plane.at[pl.ds(sid * ZCHUNK + z * ZB, ZB)], smisc)
                for z in range(12)]
            ds.append(pltpu.async_copy(
                zbuf.at[pl.ds(0, 256)],
                plane.at[pl.ds(sid * ZCHUNK + 12 * ZB, 256)], smisc))
            for d in ds:
                d.wait()

        @pl.when(sid == NS - 1)
        def _():
            ds = [pltpu.async_copy(
                zbuf, plane.at[pl.ds((NS - 1) * ZCHUNK + z * ZB, ZB)], smisc)
                for z in range(11)]
            ds.append(pltpu.async_copy(
                zbuf.at[pl.ds(0, ZLAST - 11 * ZB)],
                plane.at[pl.ds((NS - 1) * ZCHUNK + 11 * ZB, ZLAST - 11 * ZB)],
                smisc))
            for d in ds:
                d.wait()

    zero_plane()
    plsc.subcore_barrier()

    def one_image(ib, carry):
        b = cid * (B // NC) + ib

        for ch in range(NCH):
            pstart = tile_px + ch * CP
            dfx = pltpu.async_copy(flow.at[b, 0, pl.ds(pstart, CP)], ua0,
                                   smisc)
            dfy = pltpu.async_copy(flow.at[b, 1, pl.ds(pstart, CP)], ua1,
                                   smisc)
            dfx.wait()
            dfy.wait()

            def vreg(j, carry, _ch=ch, _pstart=pstart):
                p0 = _pstart + j * 16
                o = _ch * CP + j * 16
                xs = (p0 % W).astype(F32) + lane_f
                ys = jnp.full((16,), 1.0, F32) * (p0 // W).astype(F32)
                tx = jnp.clip(xs + ua0[pl.ds(j * 16, 16)], -3.0, 513.0)
                ty = jnp.clip(ys + ua1[pl.ds(j * 16, 16)], -3.0, 513.0)
                x0 = tx.astype(I32)
                y0 = ty.astype(I32)
                x0 = jnp.where(x0.astype(F32) > tx, x0 - 1, x0)
                y0 = jnp.where(y0.astype(F32) > ty, y0 - 1, y0)
                fxr[pl.ds(o, 16)] = tx - x0.astype(F32)
                fyr[pl.ds(o, 16)] = ty - y0.astype(F32)
                x0c = jnp.clip(x0, -2, 512)
                y0c = jnp.clip(y0, -2, 512)
                base = (y0c + 2) * PCOLS + (x0c + XOFF)
                ir0[pl.ds(o, 16)] = base
                return carry

            lax.fori_loop(0, NV, vreg, None, unroll=False)

        def one_channel(c, carry):
            vdesc = [None, None]
            scdesc = [[], []]
            vdesc[0] = pltpu.async_copy(
                frame.at[b, c, pl.ds(tile_px, CP)], val0, sv0)
            for ch in range(NCH):
                s = ch & 1
                for d in scdesc[s]:
                    d.wait()
                scdesc[s] = []
                vdesc[s].wait()
                if ch < NCH - 1:
                    vdesc[1 - s] = pltpu.async_copy(
                        frame.at[b, c, pl.ds(tile_px + (ch + 1) * CP, CP)],
                        vals[1 - s], svs[1 - s])
                vb = vals[s]
                u0, u1, u2, u3 = upds[s]
                j1, j2, j3 = jdx[s]

                def vreg(j, carry, _ch=ch, _vb=vb,
                         _u0=u0, _u1=u1, _u2=u2, _u3=u3,
                         _j1=j1, _j2=j2, _j3=j3):
                    o = _ch * CP + j * 16
                    cc = j * 16
                    v = _vb[pl.ds(cc, 16)]
                    fx = fxr[pl.ds(o, 16)]
                    fy = fyr[pl.ds(o, 16)]
                    bs = ir0[pl.ds(o, 16)]
                    vgy = v - v * fy
                    vfy = v * fy
                    _u0[pl.ds(cc, 16)] = vgy - vgy * fx
                    _u1[pl.ds(cc, 16)] = vgy * fx
                    _u2[pl.ds(cc, 16)] = vfy - vfy * fx
                    _u3[pl.ds(cc, 16)] = vfy * fx
                    _j1[pl.ds(cc, 16)] = bs + 1
                    _j2[pl.ds(cc, 16)] = bs + PCOLS
                    _j3[pl.ds(cc, 16)] = bs + (PCOLS + 1)
                    return carry

                lax.fori_loop(0, NV, vreg, None, unroll=False)
                off = ch * CP
                scdesc[s].append(pltpu.async_copy(
                    upds[s][0], plane.at[ir0.at[pl.ds(off, CP)]],
                    sscs[s], add=True))
                for k in range(3):
                    scdesc[s].append(pltpu.async_copy(
                        upds[s][k + 1], plane.at[jdx[s][k]],
                        sscs[s], add=True))
            for s in range(2):
                for d in scdesc[s]:
                    d.wait()
            plsc.subcore_barrier()

            ddescs = []
            for r in range(ROWS_PT):
                row = sid * ROWS_PT + r
                src_off = pl.multiple_of((row + 2) * PCOLS + XOFF, 128)
                dst_off = pl.multiple_of(row * W, W)
                ddescs.append(pltpu.async_copy(
                    plane.at[pl.ds(src_off, W)],
                    out.at[b, c, pl.ds(dst_off, W)], smisc))
            for d in ddescs:
                d.wait()
            plsc.subcore_barrier()

            zero_plane()
            plsc.subcore_barrier()
            return carry

        lax.fori_loop(0, C, one_channel, None, unroll=False)
        return carry

    lax.fori_loop(0, B // NC, one_image, None, unroll=False)


def kernel(frame, flow):
    frame2 = frame.reshape(B, C, P)
    flow2 = flow.reshape(B, 2, P)
    mesh = plsc.VectorSubcoreMesh(core_axis_name="c", subcore_axis_name="s")
    fn = pl.kernel(
        _splat_body,
        out_type=jax.ShapeDtypeStruct((B, C, P), F32),
        mesh=mesh,
        scratch_types=[
            pltpu.VMEM_SHARED((PLANE,), F32),
            pltpu.VMEM((NPT,), F32),
            pltpu.VMEM((NPT,), F32),
            pltpu.VMEM((NPT,), I32),
            pltpu.VMEM((CP,), F32),
            pltpu.VMEM((CP,), F32),
            pltpu.VMEM((CP,), F32),
            pltpu.VMEM((CP,), F32),
            pltpu.VMEM((CP,), F32),
            pltpu.VMEM((CP,), F32),
            pltpu.VMEM((CP,), F32),
            pltpu.VMEM((CP,), F32),
            pltpu.VMEM((CP,), F32),
            pltpu.VMEM((CP,), F32),
            pltpu.VMEM((CP,), I32),
            pltpu.VMEM((CP,), I32),
            pltpu.VMEM((CP,), I32),
            pltpu.VMEM((CP,), I32),
            pltpu.VMEM((CP,), I32),
            pltpu.VMEM((CP,), I32),
            pltpu.SemaphoreType.DMA,
            pltpu.SemaphoreType.DMA,
            pltpu.SemaphoreType.DMA,
            pltpu.SemaphoreType.DMA,
            pltpu.SemaphoreType.DMA,
        ],
    )
    out = fn(frame2, flow2)
    return out.reshape(B, C, H, W)


# speedup vs baseline: 11.4524x; 1.1688x over previous
import jax
import jax.numpy as jnp
from jax import lax
from jax.experimental import pallas as pl
from jax.experimental.pallas import tpu as pltpu
from jax.experimental.pallas import tpu_sc as plsc

B, C, H, W = 8, 16, 512, 512
P = H * W
NC, NS = 2, 16
NPT = P // NS
CP = 2048
NCH = NPT // CP
NV = CP // 16

PROWS = 516
PCOLS = 768
XOFF = 128
PLANE = PROWS * PCOLS
ZCHUNK = 24832
ZLAST = PLANE - (NS - 1) * ZCHUNK
ROWS_PT = H // NS

F32 = jnp.float32
I32 = jnp.int32


def _splat_body(frame, flow, out, plane,
                fxr, fyr, ir0,
                val0, val1,
                ua0, ua1, ua2, ua3, ub0, ub1, ub2, ub3,
                ja1, ja2, ja3, jb1, jb2, jb3,
                sv0, sv1, ssc0, ssc1, smisc):
    cid = lax.axis_index("c")
    sid = lax.axis_index("s")
    tile_px = sid * NPT
    lane_f = lax.iota(I32, 16).astype(F32)

    vals = (val0, val1)
    svs = (sv0, sv1)
    upds = ((ua0, ua1, ua2, ua3), (ub0, ub1, ub2, ub3))
    jdx = ((ja1, ja2, ja3), (jb1, jb2, jb3))
    sscs = (ssc0, ssc1)

    ZB = 2048
    zbuf = ua0

    def zero_plane():
        def zb_init(j, carry):
            zbuf[pl.ds(j * 16, 16)] = jnp.zeros((16,), F32)
            return carry

        lax.fori_loop(0, ZB // 16, zb_init, None, unroll=False)

        @pl.when(sid < NS - 1)
        def _():
            ds = [pltpu.async_copy(
                zbuf, plane.at[pl.ds(sid * ZCHUNK + z * ZB, ZB)], smisc)
                for z in range(12)]
            ds.append(pltpu.async_copy(
                zbuf.at[pl.ds(0, 256)],
                plane.at[pl.ds(sid * ZCHUNK + 12 * ZB, 256)], smisc))
            for d in ds:
                d.wait()

        @pl.when(sid == NS - 1)
        def _():
            ds = [pltpu.async_copy(
                zbuf, plane.at[pl.ds((NS - 1) * ZCHUNK + z * ZB, ZB)], smisc)
                for z in range(11)]
            ds.append(pltpu.async_copy(
                zbuf.at[pl.ds(0, ZLAST - 11 * ZB)],
                plane.at[pl.ds((NS - 1) * ZCHUNK + 11 * ZB, ZLAST - 11 * ZB)],
                smisc))
            for d in ds:
                d.wait()

    zero_plane()
    plsc.subcore_barrier()

    def one_image(ib, carry):
        b = cid * (B // NC) + ib

        for ch in range(NCH):
            pstart = tile_px + ch * CP
            r0 = pstart // W
            ds = []
            for r in range(CP // W):
                ds.append(pltpu.async_copy(
                    flow.at[b, 0, r0 + r], ua0.at[pl.ds(r * W, W)], smisc))
                ds.append(pltpu.async_copy(
                    flow.at[b, 1, r0 + r], ua1.at[pl.ds(r * W, W)], smisc))
            for d in ds:
                d.wait()

            def vreg(j, carry, _ch=ch, _pstart=pstart):
                p0 = _pstart + j * 16
                o = _ch * CP + j * 16
                xs = (p0 % W).astype(F32) + lane_f
                ys = jnp.full((16,), 1.0, F32) * (p0 // W).astype(F32)
                tx = jnp.clip(xs + ua0[pl.ds(j * 16, 16)], -3.0, 513.0)
                ty = jnp.clip(ys + ua1[pl.ds(j * 16, 16)], -3.0, 513.0)
                x0 = tx.astype(I32)
                y0 = ty.astype(I32)
                x0 = jnp.where(x0.astype(F32) > tx, x0 - 1, x0)
                y0 = jnp.where(y0.astype(F32) > ty, y0 - 1, y0)
                fxr[pl.ds(o, 16)] = tx - x0.astype(F32)
                fyr[pl.ds(o, 16)] = ty - y0.astype(F32)
                x0c = jnp.clip(x0, -2, 512)
                y0c = jnp.clip(y0, -2, 512)
                base = (y0c + 2) * PCOLS + (x0c + XOFF)
                ir0[pl.ds(o, 16)] = base
                return carry

            lax.fori_loop(0, NV, vreg, None, unroll=False)

        def one_channel(c, carry):
            vdesc = [None, None]
            scdesc = [[], []]
            r00 = tile_px // W
            vdesc[0] = [pltpu.async_copy(
                frame.at[b, c, r00 + r], val0.at[pl.ds(r * W, W)], sv0)
                for r in range(CP // W)]
            for ch in range(NCH):
                s = ch & 1
                for d in scdesc[s]:
                    d.wait()
                scdesc[s] = []
                for d in vdesc[s]:
                    d.wait()
                if ch < NCH - 1:
                    r1 = (tile_px + (ch + 1) * CP) // W
                    vdesc[1 - s] = [pltpu.async_copy(
                        frame.at[b, c, r1 + r], vals[1 - s].at[pl.ds(r * W, W)],
                        svs[1 - s]) for r in range(CP // W)]
                vb = vals[s]
                u0, u1, u2, u3 = upds[s]
                j1, j2, j3 = jdx[s]

                def vreg(j, carry, _ch=ch, _vb=vb,
                         _u0=u0, _u1=u1, _u2=u2, _u3=u3,
                         _j1=j1, _j2=j2, _j3=j3):
                    o = _ch * CP + j * 16
                    cc = j * 16
                    v = _vb[pl.ds(cc, 16)]
                    fx = fxr[pl.ds(o, 16)]
                    fy = fyr[pl.ds(o, 16)]
                    bs = ir0[pl.ds(o, 16)]
                    vgy = v - v * fy
                    vfy = v * fy
                    _u0[pl.ds(cc, 16)] = vgy - vgy * fx
                    _u1[pl.ds(cc, 16)] = vgy * fx
                    _u2[pl.ds(cc, 16)] = vfy - vfy * fx
                    _u3[pl.ds(cc, 16)] = vfy * fx
                    _j1[pl.ds(cc, 16)] = bs + 1
                    _j2[pl.ds(cc, 16)] = bs + PCOLS
                    _j3[pl.ds(cc, 16)] = bs + (PCOLS + 1)
                    return carry

                lax.fori_loop(0, NV, vreg, None, unroll=False)
                off = ch * CP
                scdesc[s].append(pltpu.async_copy(
                    upds[s][0], plane.at[ir0.at[pl.ds(off, CP)]],
                    sscs[s], add=True))
                for k in range(3):
                    scdesc[s].append(pltpu.async_copy(
                        upds[s][k + 1], plane.at[jdx[s][k]],
                        sscs[s], add=True))
            for s in range(2):
                for d in scdesc[s]:
                    d.wait()
            plsc.subcore_barrier()

            ddescs = []
            for r in range(ROWS_PT):
                row = sid * ROWS_PT + r
                src_off = pl.multiple_of((row + 2) * PCOLS + XOFF, 128)
                ddescs.append(pltpu.async_copy(
                    plane.at[pl.ds(src_off, W)],
                    out.at[b, c, row], smisc))
            for d in ddescs:
                d.wait()
            plsc.subcore_barrier()

            zero_plane()
            plsc.subcore_barrier()
            return carry

        lax.fori_loop(0, C, one_channel, None, unroll=False)
        return carry

    lax.fori_loop(0, B // NC, one_image, None, unroll=False)


def kernel(frame, flow):
    mesh = plsc.VectorSubcoreMesh(core_axis_name="c", subcore_axis_name="s")
    fn = pl.kernel(
        _splat_body,
        out_type=jax.ShapeDtypeStruct((B, C, H, W), F32),
        mesh=mesh,
        scratch_types=[
            pltpu.VMEM_SHARED((PLANE,), F32),
            pltpu.VMEM((NPT,), F32),
            pltpu.VMEM((NPT,), F32),
            pltpu.VMEM((NPT,), I32),
            pltpu.VMEM((CP,), F32),
            pltpu.VMEM((CP,), F32),
            pltpu.VMEM((CP,), F32),
            pltpu.VMEM((CP,), F32),
            pltpu.VMEM((CP,), F32),
            pltpu.VMEM((CP,), F32),
            pltpu.VMEM((CP,), F32),
            pltpu.VMEM((CP,), F32),
            pltpu.VMEM((CP,), F32),
            pltpu.VMEM((CP,), F32),
            pltpu.VMEM((CP,), I32),
            pltpu.VMEM((CP,), I32),
            pltpu.VMEM((CP,), I32),
            pltpu.VMEM((CP,), I32),
            pltpu.VMEM((CP,), I32),
            pltpu.VMEM((CP,), I32),
            pltpu.SemaphoreType.DMA,
            pltpu.SemaphoreType.DMA,
            pltpu.SemaphoreType.DMA,
            pltpu.SemaphoreType.DMA,
            pltpu.SemaphoreType.DMA,
        ],
    )
    return fn(frame, flow)
